# baseline (speedup 1.0000x reference)
import functools

import jax
import jax.numpy as jnp
from jax import lax
from jax.experimental import pallas as pl
from jax.experimental.pallas import tpu as pltpu
from jax.experimental.pallas import tpu_sc as plsc

NUM_CORES = 2
NUM_SUBCORES = 16
NUM_WORKERS = NUM_CORES * NUM_SUBCORES
LANES = 16

BATCH = 16384
DIM = 64
B_PER_W = BATCH // NUM_WORKERS
CHUNK = 128
N_CHUNKS = B_PER_W // CHUNK


def _body(users_hbm, items_hbm, utab_hbm, itab_hbm, out_hbm,
          idx_u, idx_i, u_rows, v_rows, out_v, sem):
    wid = lax.axis_index("s") * NUM_CORES + lax.axis_index("c")
    base = wid * B_PER_W

    for c in range(N_CHUNKS):
        pltpu.sync_copy(users_hbm.at[pl.ds(base + c * CHUNK, CHUNK)], idx_u.at[c])
        pltpu.sync_copy(items_hbm.at[pl.ds(base + c * CHUNK, CHUNK)], idx_i.at[c])

    copies = []
    for c in range(N_CHUNKS):
        copies.append(pltpu.async_copy(
            utab_hbm.at[idx_u.at[c]], u_rows.at[pl.ds(c * CHUNK, CHUNK), :], sem))
        copies.append(pltpu.async_copy(
            itab_hbm.at[idx_i.at[c]], v_rows.at[pl.ds(c * CHUNK, CHUNK), :], sem))
    for cp in copies:
        cp.wait()

    lanes = lax.iota(jnp.int32, LANES)
    perms = [lanes ^ step for step in (8, 4, 2, 1)]

    def group(g, _):
        vec = jnp.zeros((LANES,), jnp.float32)
        for j in range(LANES):
            r = g * LANES + j
            acc = u_rows[r, pl.ds(0, LANES)] * v_rows[r, pl.ds(0, LANES)]
            for k in range(1, DIM // LANES):
                acc += (u_rows[r, pl.ds(k * LANES, LANES)]
                        * v_rows[r, pl.ds(k * LANES, LANES)])
            for p in perms:
                acc = acc + acc.at[p].get(mode="promise_in_bounds")
            vec = jnp.where(lanes == j, acc, vec)
        out_v[pl.ds(g * LANES, LANES)] = 1.0 / (1.0 + jnp.exp(-vec))
        return _

    lax.fori_loop(0, B_PER_W // LANES, group, None)

    pltpu.sync_copy(out_v, out_hbm.at[pl.ds(base, B_PER_W)])


@functools.partial(jax.jit, donate_argnums=())
def _run(users, items, user_table, item_table):
    mesh = plsc.VectorSubcoreMesh(core_axis_name="c", subcore_axis_name="s")
    return pl.kernel(
        _body,
        out_type=jax.ShapeDtypeStruct((BATCH,), jnp.float32),
        mesh=mesh,
        compiler_params=pltpu.CompilerParams(use_tc_tiling_on_sc=False),
        scratch_types=[
            pltpu.VMEM((N_CHUNKS, CHUNK), jnp.int32),
            pltpu.VMEM((N_CHUNKS, CHUNK), jnp.int32),
            pltpu.VMEM((B_PER_W, DIM), jnp.float32),
            pltpu.VMEM((B_PER_W, DIM), jnp.float32),
            pltpu.VMEM((B_PER_W,), jnp.float32),
            pltpu.SemaphoreType.DMA,
        ],
    )(users, items, user_table, item_table)


def kernel(users, items, user_table, item_table):
    return _run(users, items, user_table, item_table)
